# Initial kernel scaffold; baseline (speedup 1.0000x reference)
import jax
import jax.numpy as jnp
from jax.experimental import pallas as pl


def kernel(inputs):
    raise NotImplementedError("write your pallas kernel here")


# speedup vs baseline: 40.0476x; 40.0476x over previous
import jax
import jax.numpy as jnp
from jax.experimental import pallas as pl

K = 8
BR = 64


def _topk_body(x_ref, o_ref):
    x = x_ref[0]
    neg = jnp.float32(-jnp.inf)
    t = jnp.full((BR, 1), jnp.inf, jnp.float32)
    cum = jnp.zeros((BR, 1), jnp.int32)
    out = jnp.zeros((BR, K), jnp.float32)
    jidx = jax.lax.broadcasted_iota(jnp.int32, (BR, K), 1)
    for _ in range(K):
        xm = jnp.where(x < t, x, neg)
        m = jnp.max(xm, axis=-1, keepdims=True)
        c = jnp.sum((x == m).astype(jnp.int32), axis=-1, keepdims=True)
        sel = (jidx >= cum) & (jidx < cum + c)
        out = jnp.where(sel, m, out)
        cum = cum + c
        t = m
    o_ref[0] = out


def kernel(inputs):
    B, D, N = inputs.shape
    out = pl.pallas_call(
        _topk_body,
        grid=(B, D // BR),
        in_specs=[pl.BlockSpec((1, BR, N), lambda b, r: (b, r, 0))],
        out_specs=pl.BlockSpec((1, BR, K), lambda b, r: (b, r, 0)),
        out_shape=jax.ShapeDtypeStruct((B, D, K), jnp.float32),
    )(inputs)
    return jnp.transpose(out, (0, 2, 1))
